# Initial kernel scaffold; baseline (speedup 1.0000x reference)
import jax
import jax.numpy as jnp
from jax.experimental import pallas as pl


def kernel(features, indices, W1, b1, W2, b2):
    raise NotImplementedError("write your pallas kernel here")


# speedup vs baseline: 15.6752x; 15.6752x over previous
import functools

import jax
import jax.numpy as jnp
from jax import lax
from jax.experimental import pallas as pl
from jax.experimental.pallas import tpu as pltpu
from jax.experimental.pallas import tpu_sc as plsc

N = 10000
DEG = 32
D = 128
H = 64
E = N * DEG
NP = 10240
CHUNK = 128
NCHUNK = NP // CHUNK
K_KEEP = 16


def _node_mlp_body(f_ref, wcat_ref, b1_ref, a_ref, b_ref):
    ab = jnp.dot(f_ref[...].astype(jnp.bfloat16),
                 wcat_ref[...].astype(jnp.bfloat16),
                 preferred_element_type=jnp.float32)
    a_ref[...] = ab[:, :H] + b1_ref[...]
    b_ref[...] = jnp.concatenate([ab[:, H:], jnp.zeros_like(ab[:, H:])], axis=1)


def _node_mlp(feat_p, wcat, b1r):
    blk = 1024
    return pl.pallas_call(
        _node_mlp_body,
        grid=(NP // blk,),
        in_specs=[
            pl.BlockSpec((blk, D), lambda i: (i, 0)),
            pl.BlockSpec((D, 2 * H), lambda i: (0, 0)),
            pl.BlockSpec((1, H), lambda i: (0, 0)),
        ],
        out_specs=[
            pl.BlockSpec((blk, H), lambda i: (i, 0)),
            pl.BlockSpec((blk, 2 * H), lambda i: (i, 0)),
        ],
        out_shape=[
            jax.ShapeDtypeStruct((NP, H), jnp.float32),
            jax.ShapeDtypeStruct((NP, 2 * H), jnp.float32),
        ],
    )(feat_p, wcat, b1r)


def _sc_gather_body(b_hbm, colt_hbm, bgt_hbm, colv, rows, sem):
    info = plsc.get_sparse_core_info()
    j = lax.axis_index("s") * info.num_cores + lax.axis_index("c")

    def step(t, carry):
        i0 = t * CHUNK
        pltpu.sync_copy(colt_hbm.at[j, pl.ds(i0, CHUNK)], colv)
        pltpu.async_copy(b_hbm.at[colv], rows, sem).wait()
        pltpu.sync_copy(rows, bgt_hbm.at[j, pl.ds(i0, CHUNK), :])
        return carry

    lax.fori_loop(0, NCHUNK, step, 0)


def _sc_gather(b_nodes, colt):
    mesh = plsc.VectorSubcoreMesh(core_axis_name="c", subcore_axis_name="s")
    kern = functools.partial(
        pl.kernel,
        out_type=jax.ShapeDtypeStruct((DEG, NP, 2 * H), jnp.float32),
        mesh=mesh,
        scratch_types=[
            pltpu.VMEM((CHUNK,), jnp.int32),
            pltpu.VMEM((CHUNK, 2 * H), jnp.float32),
            pltpu.SemaphoreType.DMA,
        ],
    )(_sc_gather_body)
    return kern(b_nodes, colt)


def _edge_score_body(bgt_ref, a_ref, w2_ref, b2_ref, zt_ref, ssq_ref):
    b = pl.program_id(0)
    q = jnp.maximum(bgt_ref[..., :H] + a_ref[...][None, :, :], 0.0)
    qb = q.astype(jnp.bfloat16).astype(jnp.float32)
    w2b = w2_ref[...].astype(jnp.bfloat16).astype(jnp.float32)
    z = jnp.sum(qb * w2b.reshape(1, 1, H), axis=2) + b2_ref[...]
    node = b * CHUNK + lax.broadcasted_iota(jnp.int32, (DEG, CHUNK), 1)
    z = jnp.where(node < N, z, 0.0)
    zt_ref[...] = z

    @pl.when(b == 0)
    def _():
        ssq_ref[...] = jnp.zeros((1, 1), jnp.float32)

    ssq_ref[...] += jnp.sum(z * z)[None, None]


def _edge_scores(bgt, a_nodes, w2r, b2r):
    return pl.pallas_call(
        _edge_score_body,
        grid=(NCHUNK,),
        in_specs=[
            pl.BlockSpec((DEG, CHUNK, 2 * H), lambda b: (0, b, 0)),
            pl.BlockSpec((CHUNK, H), lambda b: (b, 0)),
            pl.BlockSpec((1, H), lambda b: (0, 0)),
            pl.BlockSpec((1, 1), lambda b: (0, 0)),
        ],
        out_specs=[
            pl.BlockSpec((DEG, CHUNK), lambda b: (0, b)),
            pl.BlockSpec((1, 1), lambda b: (0, 0)),
        ],
        out_shape=[
            jax.ShapeDtypeStruct((DEG, NP), jnp.float32),
            jax.ShapeDtypeStruct((1, 1), jnp.float32),
        ],
    )(bgt, a_nodes, w2r, b2r)


def _topk_body(zt_ref, colt_ref, ssq_ref, out_ref):
    blk = zt_ref.shape[1]
    s = 1.0 / jnp.maximum(jnp.sqrt(ssq_ref[...]), 1e-12)
    zz = zt_ref[...] * s
    m = jnp.max(zz, axis=0, keepdims=True)
    ex = jnp.exp(zz - m)
    den = jnp.sum(ex, axis=0, keepdims=True)
    pi = ex / den
    cnt = jnp.zeros((DEG, blk), jnp.float32)
    for bb in range(DEG):
        cnt += (pi[bb:bb + 1, :] > pi).astype(jnp.float32)
    thr = jnp.min(jnp.where(cnt <= float(K_KEEP - 1), pi, jnp.inf),
                  axis=0, keepdims=True)
    masked = jnp.where(pi - thr + 1e-15 > 0.0, pi, 0.0)
    col = colt_ref[...]
    rank = jnp.zeros((DEG, blk), jnp.int32)
    for bb in range(DEG):
        rank += (col[bb:bb + 1, :] < col).astype(jnp.int32)
    rows_iota = lax.broadcasted_iota(jnp.int32, (DEG, blk), 0)
    out = jnp.zeros((DEG, blk), jnp.float32)
    for jj in range(DEG):
        out += jnp.where(rank[jj:jj + 1, :] == rows_iota,
                         masked[jj:jj + 1, :], 0.0)
    out_ref[...] = out


def _topk_mask(zt, colt, ssq):
    blk = 1024
    return pl.pallas_call(
        _topk_body,
        grid=(NP // blk,),
        in_specs=[
            pl.BlockSpec((DEG, blk), lambda i: (0, i)),
            pl.BlockSpec((DEG, blk), lambda i: (0, i)),
            pl.BlockSpec((1, 1), lambda i: (0, 0)),
        ],
        out_specs=pl.BlockSpec((DEG, blk), lambda i: (0, i)),
        out_shape=jax.ShapeDtypeStruct((DEG, NP), jnp.float32),
    )(zt, colt, ssq)


def kernel(features, indices, W1, b1, W2, b2):
    col = indices[1]
    wcat = jnp.concatenate([W1[:, :D].T, W1[:, D:].T], axis=1)
    b1r = b1.reshape(1, H)
    w2r = W2.reshape(1, H)
    b2r = b2.reshape(1, 1)
    feat_p = jnp.zeros((NP, D), jnp.float32).at[:N].set(features)
    colt = (jnp.zeros((NP, DEG), jnp.int32)
            .at[:N].set(col.reshape(N, DEG)).T)

    a_nodes, b_nodes = _node_mlp(feat_p, wcat, b1r)
    bgt = _sc_gather(b_nodes, colt)
    zt, ssq = _edge_scores(bgt, a_nodes, w2r, b2r)
    out_t = _topk_mask(zt, colt, ssq)
    return out_t[:, :N].T.reshape(E)


# speedup vs baseline: 18.0845x; 1.1537x over previous
import functools

import jax
import jax.numpy as jnp
from jax import lax
from jax.experimental import pallas as pl
from jax.experimental.pallas import tpu as pltpu
from jax.experimental.pallas import tpu_sc as plsc

N = 10000
DEG = 32
D = 128
H = 64
E = N * DEG
NP = 10240
CHUNK = 128
NCHUNK = NP // CHUNK
K_KEEP = 16


def _node_mlp_body(f_ref, wcat_ref, b1_ref, a_ref, b_ref):
    ab = jnp.dot(f_ref[...].astype(jnp.bfloat16),
                 wcat_ref[...].astype(jnp.bfloat16),
                 preferred_element_type=jnp.float32)
    a_ref[...] = ab[:, :H] + b1_ref[...]
    b_ref[...] = jnp.concatenate([ab[:, H:], jnp.zeros_like(ab[:, H:])], axis=1)


def _node_mlp(feat_p, wcat, b1r):
    blk = 1024
    return pl.pallas_call(
        _node_mlp_body,
        grid=(NP // blk,),
        in_specs=[
            pl.BlockSpec((blk, D), lambda i: (i, 0)),
            pl.BlockSpec((D, 2 * H), lambda i: (0, 0)),
            pl.BlockSpec((1, H), lambda i: (0, 0)),
        ],
        out_specs=[
            pl.BlockSpec((blk, H), lambda i: (i, 0)),
            pl.BlockSpec((blk, 2 * H), lambda i: (i, 0)),
        ],
        out_shape=[
            jax.ShapeDtypeStruct((NP, H), jnp.float32),
            jax.ShapeDtypeStruct((NP, 2 * H), jnp.float32),
        ],
    )(feat_p, wcat, b1r)


def _sc_gather_body(b_hbm, colt_hbm, bgt_hbm, colrow, rows0, rows1,
                    sem0, sem1):
    info = plsc.get_sparse_core_info()
    j = lax.axis_index("s") * info.num_cores + lax.axis_index("c")
    pltpu.sync_copy(colt_hbm.at[j], colrow)

    def _gather(t, buf, sem):
        idx = colrow.at[pl.ds(t * CHUNK, CHUNK)]
        return pltpu.make_async_copy(b_hbm.at[idx], buf, sem)

    def _write(t, buf):
        pltpu.sync_copy(buf, bgt_hbm.at[j, pl.ds(t * CHUNK, CHUNK), :])

    _gather(0, rows0, sem0).start()

    def step(p, carry):
        t0 = 2 * p
        _gather(t0 + 1, rows1, sem1).start()
        _gather(t0, rows0, sem0).wait()
        _write(t0, rows0)

        @pl.when(p < NCHUNK // 2 - 1)
        def _():
            _gather(t0 + 2, rows0, sem0).start()

        _gather(t0 + 1, rows1, sem1).wait()
        _write(t0 + 1, rows1)
        return carry

    lax.fori_loop(0, NCHUNK // 2, step, 0)


def _sc_gather(b_nodes, colt):
    mesh = plsc.VectorSubcoreMesh(core_axis_name="c", subcore_axis_name="s")
    kern = functools.partial(
        pl.kernel,
        out_type=jax.ShapeDtypeStruct((DEG, NP, 2 * H), jnp.float32),
        mesh=mesh,
        scratch_types=[
            pltpu.VMEM((NP,), jnp.int32),
            pltpu.VMEM((CHUNK, 2 * H), jnp.float32),
            pltpu.VMEM((CHUNK, 2 * H), jnp.float32),
            pltpu.SemaphoreType.DMA,
            pltpu.SemaphoreType.DMA,
        ],
    )(_sc_gather_body)
    return kern(b_nodes, colt)


def _edge_score_body(bgt_ref, a_ref, w2_ref, b2_ref, zt_ref, ssq_ref):
    b = pl.program_id(0)
    q = jnp.maximum(bgt_ref[..., :H] + a_ref[...][None, :, :], 0.0)
    qb = q.astype(jnp.bfloat16).astype(jnp.float32)
    w2b = w2_ref[...].astype(jnp.bfloat16).astype(jnp.float32)
    z = jnp.sum(qb * w2b.reshape(1, 1, H), axis=2) + b2_ref[...]
    node = b * CHUNK + lax.broadcasted_iota(jnp.int32, (DEG, CHUNK), 1)
    z = jnp.where(node < N, z, 0.0)
    zt_ref[...] = z

    @pl.when(b == 0)
    def _():
        ssq_ref[...] = jnp.zeros((1, 1), jnp.float32)

    ssq_ref[...] += jnp.sum(z * z)[None, None]


def _edge_scores(bgt, a_nodes, w2r, b2r):
    return pl.pallas_call(
        _edge_score_body,
        grid=(NCHUNK,),
        in_specs=[
            pl.BlockSpec((DEG, CHUNK, 2 * H), lambda b: (0, b, 0)),
            pl.BlockSpec((CHUNK, H), lambda b: (b, 0)),
            pl.BlockSpec((1, H), lambda b: (0, 0)),
            pl.BlockSpec((1, 1), lambda b: (0, 0)),
        ],
        out_specs=[
            pl.BlockSpec((DEG, CHUNK), lambda b: (0, b)),
            pl.BlockSpec((1, 1), lambda b: (0, 0)),
        ],
        out_shape=[
            jax.ShapeDtypeStruct((DEG, NP), jnp.float32),
            jax.ShapeDtypeStruct((1, 1), jnp.float32),
        ],
    )(bgt, a_nodes, w2r, b2r)


def _topk_body(zt_ref, colt_ref, ssq_ref, out_ref):
    blk = zt_ref.shape[1]
    s = 1.0 / jnp.maximum(jnp.sqrt(ssq_ref[...]), 1e-12)
    zz = zt_ref[...] * s
    m = jnp.max(zz, axis=0, keepdims=True)
    ex = jnp.exp(zz - m)
    den = jnp.sum(ex, axis=0, keepdims=True)
    pi = ex / den
    cnt = jnp.zeros((DEG, blk), jnp.float32)
    for bb in range(DEG):
        cnt += (pi[bb:bb + 1, :] > pi).astype(jnp.float32)
    thr = jnp.min(jnp.where(cnt <= float(K_KEEP - 1), pi, jnp.inf),
                  axis=0, keepdims=True)
    masked = jnp.where(pi - thr + 1e-15 > 0.0, pi, 0.0)
    col = colt_ref[...]
    rank = jnp.zeros((DEG, blk), jnp.int32)
    for bb in range(DEG):
        rank += (col[bb:bb + 1, :] < col).astype(jnp.int32)
    rows_iota = lax.broadcasted_iota(jnp.int32, (DEG, blk), 0)
    out = jnp.zeros((DEG, blk), jnp.float32)
    for jj in range(DEG):
        out += jnp.where(rank[jj:jj + 1, :] == rows_iota,
                         masked[jj:jj + 1, :], 0.0)
    out_ref[...] = out


def _topk_mask(zt, colt, ssq):
    blk = 1024
    return pl.pallas_call(
        _topk_body,
        grid=(NP // blk,),
        in_specs=[
            pl.BlockSpec((DEG, blk), lambda i: (0, i)),
            pl.BlockSpec((DEG, blk), lambda i: (0, i)),
            pl.BlockSpec((1, 1), lambda i: (0, 0)),
        ],
        out_specs=pl.BlockSpec((DEG, blk), lambda i: (0, i)),
        out_shape=jax.ShapeDtypeStruct((DEG, NP), jnp.float32),
    )(zt, colt, ssq)


def kernel(features, indices, W1, b1, W2, b2):
    col = indices[1]
    wcat = jnp.concatenate([W1[:, :D].T, W1[:, D:].T], axis=1)
    b1r = b1.reshape(1, H)
    w2r = W2.reshape(1, H)
    b2r = b2.reshape(1, 1)
    feat_p = jnp.zeros((NP, D), jnp.float32).at[:N].set(features)
    colt = (jnp.zeros((NP, DEG), jnp.int32)
            .at[:N].set(col.reshape(N, DEG)).T)

    a_nodes, b_nodes = _node_mlp(feat_p, wcat, b1r)
    bgt = _sc_gather(b_nodes, colt)
    zt, ssq = _edge_scores(bgt, a_nodes, w2r, b2r)
    out_t = _topk_mask(zt, colt, ssq)
    return out_t[:, :N].T.reshape(E)
